# baseline (speedup 1.0000x reference)
import functools
import jax
import jax.numpy as jnp
from jax import lax
from jax.experimental import pallas as pl
from jax.experimental.pallas import tpu as pltpu
from jax.experimental.pallas import tpu_sc as plsc

_N_VOCAB = 1000
_HIDDEN = 128
_B = 1024
_T = 200

_NW = 32
_N = _B * _T
_RPW = _B // _NW
_NCH = 2 * _RPW
_C0, _C1 = 128, _T - 128
_NBUF = 8
_AHEAD = 4
_NGRP = _NCH // _NBUF


def _gather_sc(x, emb):
    mesh = plsc.VectorSubcoreMesh(core_axis_name="c", subcore_axis_name="s")
    sizes = [_C0 if k % 2 == 0 else _C1 for k in range(_NBUF)]
    offs = [0 if k % 2 == 0 else _C0 for k in range(_NBUF)]

    @functools.partial(
        pl.kernel,
        mesh=mesh,
        out_type=jax.ShapeDtypeStruct((_N, _HIDDEN), jnp.float32),
        scratch_types=(
            [pltpu.VMEM((_RPW, _T), jnp.int32),
             pltpu.VMEM_SHARED((_N_VOCAB, _HIDDEN), jnp.float32)]
            + [pltpu.VMEM((sizes[k], _HIDDEN), jnp.float32) for k in range(_NBUF)]
            + [pltpu.SemaphoreType.DMA] * (2 * _NBUF)
        ),
    )
    def k(x_hbm, emb_hbm, out_hbm, idx_v, emb_sh, *bufs_sems):
        rows = bufs_sems[:_NBUF]
        gsem = bufs_sems[_NBUF:2 * _NBUF]
        osem = bufs_sems[2 * _NBUF:]

        sid = lax.axis_index("s")
        wid = sid * 2 + lax.axis_index("c")
        rowbase = wid * _RPW
        outbase = rowbase * _T

        @pl.when(sid == 0)
        def _():
            pltpu.sync_copy(emb_hbm, emb_sh)

        pltpu.sync_copy(x_hbm.at[pl.ds(rowbase, _RPW)], idx_v)
        plsc.subcore_barrier()

        def idx_slice(m, k):
            return idx_v.at[m // 2, pl.ds(offs[k % _NBUF], sizes[k % _NBUF])]

        def out_slice(m, k):
            return out_hbm.at[
                pl.ds(outbase + (m // 2) * _T + offs[k % _NBUF],
                      sizes[k % _NBUF])]

        def fire_gather(m, k):
            src = emb_hbm if (k % _NBUF) == 3 else emb_sh
            pltpu.make_async_copy(src.at[idx_slice(m, k)],
                                  rows[k % _NBUF], gsem[k % _NBUF]).start()

        for m in range(_AHEAD):
            fire_gather(m, m)

        def outer(g, carry):
            for k in range(_NBUF):
                m = g * _NBUF + k
                mn = m + _AHEAD
                kn = (k + _AHEAD) % _NBUF

                @pl.when(jnp.logical_and(mn < _NCH, m >= _AHEAD))
                def _():
                    pltpu.make_async_copy(rows[kn], out_slice(m - _AHEAD, kn),
                                          osem[kn]).wait()

                @pl.when(mn < _NCH)
                def _():
                    fire_gather(mn, kn)

                src = emb_hbm if k == 3 else emb_sh
                pltpu.make_async_copy(src.at[idx_slice(m, k)],
                                      rows[k], gsem[k]).wait()
                pltpu.make_async_copy(rows[k], out_slice(m, k),
                                      osem[k]).start()
            return carry

        lax.fori_loop(0, _NGRP, outer, 0)

        for k in range(_NBUF):
            m = _NCH - _NBUF + k
            pltpu.make_async_copy(rows[k], out_slice(m, k), osem[k]).wait()

    return k(x, emb)


def _mask_tc(x_lengths):
    def mask_kernel(len_ref, out_ref):
        t_idx = lax.broadcasted_iota(jnp.int32, (_B, _T), 1)
        lens = len_ref[...].reshape(_B, 1)
        out_ref[...] = (t_idx < lens).astype(jnp.float32)

    m = pl.pallas_call(
        mask_kernel,
        out_shape=jax.ShapeDtypeStruct((_B, _T), jnp.float32),
    )(x_lengths.reshape(_B, 1))
    return m[:, None, :]


def kernel(x, x_lengths, emb):
    x_emb = _gather_sc(x.astype(jnp.int32), emb).reshape(_B, _T, _HIDDEN)
    x_mask = _mask_tc(x_lengths)
    return (x_mask, x_emb)


# speedup vs baseline: 1.0673x; 1.0673x over previous
import functools
import jax
import jax.numpy as jnp
from jax import lax
from jax.experimental import pallas as pl
from jax.experimental.pallas import tpu as pltpu
from jax.experimental.pallas import tpu_sc as plsc

_N_VOCAB = 1000
_HIDDEN = 128
_B = 1024
_T = 200

_NW = 32
_N = _B * _T
_NPW = _N // _NW
_C = 128
_NCHUNK = _NPW // _C
_NBUF = 6
_AHEAD = 3


def _gather_sc(x3, emb):
    mesh = plsc.VectorSubcoreMesh(core_axis_name="c", subcore_axis_name="s")

    @functools.partial(
        pl.kernel,
        mesh=mesh,
        out_type=jax.ShapeDtypeStruct((_N, _HIDDEN), jnp.float32),
        scratch_types=[
            pltpu.VMEM((_NCHUNK, _C), jnp.int32),
            pltpu.VMEM_SHARED((_N_VOCAB, _HIDDEN), jnp.float32),
            pltpu.VMEM((_NBUF, _C, _HIDDEN), jnp.float32),
            pltpu.SemaphoreType.DMA((_NBUF,)),
            pltpu.SemaphoreType.DMA((_NBUF,)),
        ],
    )
    def k(x_hbm, emb_hbm, out_hbm, idx_v, emb_sh, rows, gsem, osem):
        sid = lax.axis_index("s")
        wid = sid * 2 + lax.axis_index("c")
        base = wid * _NCHUNK

        @pl.when(sid == 0)
        def _():
            pltpu.sync_copy(emb_hbm, emb_sh)

        pltpu.sync_copy(x_hbm.at[wid], idx_v)
        plsc.subcore_barrier()

        def out_slice(m):
            return out_hbm.at[pl.ds((base + m) * _C, _C)]

        def fire_gather(m, slot):
            pltpu.make_async_copy(emb_sh.at[idx_v.at[m]], rows.at[slot],
                                  gsem.at[slot]).start()

        for m in range(_AHEAD):
            fire_gather(m, m)

        def body(m, carry):
            slot = lax.rem(m, _NBUF)
            mn = m + _AHEAD
            slotn = lax.rem(mn, _NBUF)

            @pl.when(jnp.logical_and(mn < _NCHUNK, mn >= _NBUF))
            def _():
                pltpu.make_async_copy(rows.at[slotn], out_slice(mn - _NBUF),
                                      osem.at[slotn]).wait()

            @pl.when(mn < _NCHUNK)
            def _():
                fire_gather(mn, slotn)

            pltpu.make_async_copy(emb_sh.at[idx_v.at[m]], rows.at[slot],
                                  gsem.at[slot]).wait()
            pltpu.make_async_copy(rows.at[slot], out_slice(m),
                                  osem.at[slot]).start()
            return carry

        lax.fori_loop(0, _NCHUNK, body, 0)

        def drain(i, carry):
            m = _NCHUNK - _NBUF + i
            slot = lax.rem(m, _NBUF)
            pltpu.make_async_copy(rows.at[slot], out_slice(m),
                                  osem.at[slot]).wait()
            return carry

        lax.fori_loop(0, _NBUF, drain, 0)

    return k(x3, emb)


def _mask_tc(x_lengths):
    def mask_kernel(len_ref, out_ref):
        t_idx = lax.broadcasted_iota(jnp.int32, (_B, _T), 1)
        lens = len_ref[...].reshape(_B, 1)
        out_ref[...] = (t_idx < lens).astype(jnp.float32)

    m = pl.pallas_call(
        mask_kernel,
        out_shape=jax.ShapeDtypeStruct((_B, _T), jnp.float32),
    )(x_lengths.reshape(_B, 1))
    return m[:, None, :]


def kernel(x, x_lengths, emb):
    x3 = x.reshape(_NW, _NCHUNK, _C).astype(jnp.int32)
    x_emb = _gather_sc(x3, emb).reshape(_B, _T, _HIDDEN)
    x_mask = _mask_tc(x_lengths)
    return (x_mask, x_emb)
